# baseline (speedup 1.0000x reference)
import functools

import jax
import jax.numpy as jnp
from jax import lax
from jax.experimental import pallas as pl
from jax.experimental.pallas import tpu as pltpu
from jax.experimental.pallas import tpu_sc as plsc

EMB = 32
BATCH = 16384
CH = 128


def _make_kernel(nprows, batch):
    info = plsc.get_sparse_core_info()
    nw = info.num_cores * info.num_subcores
    b_per_w = batch // nw
    nch = (2 * b_per_w) // CH
    mesh = plsc.VectorSubcoreMesh(core_axis_name="c", subcore_axis_name="s")

    @functools.partial(
        pl.kernel,
        mesh=mesh,
        out_type=[
            jax.ShapeDtypeStruct((batch // 8, 8, EMB), jnp.float32),
            jax.ShapeDtypeStruct((batch // 8, 8, EMB), jnp.float32),
        ],
        scratch_types=[
            pltpu.VMEM((2 * b_per_w,), jnp.int32),
            pltpu.VMEM((2 * b_per_w,), jnp.int32),
            pltpu.VMEM((CH, 128), jnp.float32),
            pltpu.VMEM((CH, 128), jnp.float32),
            pltpu.VMEM((CH // 8, 8, EMB), jnp.float32),
            pltpu.SemaphoreType.DMA,
            pltpu.SemaphoreType.DMA,
        ],
        compiler_params=pltpu.CompilerParams(needs_layout_passes=False),
    )
    def gather_kernel(table_hbm, uidx_hbm, iidx_hbm, out_u, out_i,
                      idx_v, pidx_v, buf_a, buf_b, sel_v, sem_a, sem_b):
        wid = lax.axis_index("s") * info.num_cores + lax.axis_index("c")
        base = wid * b_per_w
        pltpu.sync_copy(uidx_hbm.at[pl.ds(base, b_per_w)],
                        idx_v.at[pl.ds(0, b_per_w)])
        pltpu.sync_copy(iidx_hbm.at[pl.ds(base, b_per_w)],
                        idx_v.at[pl.ds(b_per_w, b_per_w)])

        def to_packed(v, _):
            pidx_v[pl.ds(v * 16, 16)] = idx_v[pl.ds(v * 16, 16)] >> 2
            return 0

        lax.fori_loop(0, (2 * b_per_w) // 16, to_packed, 0)

        bufs = (buf_a, buf_b)
        sems = (sem_a, sem_b)

        def issue(c):
            pltpu.async_copy(table_hbm.at[pidx_v.at[pl.ds(c * CH, CH)]],
                             bufs[c % 2], sems[c % 2])

        def wait_chunk(c):
            pltpu.make_async_copy(table_hbm.at[pidx_v.at[pl.ds(c * CH, CH)]],
                                  bufs[c % 2], sems[c % 2]).wait()

        def select(c):
            buf = bufs[c % 2]
            c_base = c * CH

            def blk(b, _):
                row16 = lax.broadcasted_iota(jnp.int32, (16,), 0) + b * 16
                sub16 = (idx_v[pl.ds(c_base + b * 16, 16)] & 3) << 5
                o_hi = row16 >> 3
                o_lo = row16 & 7
                for j in range(EMB):
                    jv = jnp.full((16,), j, jnp.int32)
                    vals = plsc.load_gather(buf, [row16, sub16 + j])
                    plsc.store_scatter(sel_v, [o_hi, o_lo, jv], vals)
                return 0

            lax.fori_loop(0, CH // 16, blk, 0)

        def writeout(c):
            out = out_u if c < nch // 2 else out_i
            grp0 = (base + (c % (nch // 2)) * CH) // 8
            pltpu.sync_copy(sel_v, out.at[pl.ds(grp0, CH // 8)])

        issue(0)
        issue(1)
        for c in range(nch):
            wait_chunk(c)
            select(c)
            writeout(c)
            if c + 2 < nch:
                issue(c + 2)

    return gather_kernel


def kernel(embeddings, user_ids, item_ids):
    vocab, emb = embeddings.shape
    table2 = embeddings.reshape(vocab // 4, emb * 4)
    batch = user_ids.shape[0]
    fn = _make_kernel(vocab // 4, batch)
    users_emb, items_emb = fn(table2, user_ids, item_ids)
    return (users_emb.reshape(batch, emb), items_emb.reshape(batch, emb))


# speedup vs baseline: 2.7900x; 2.7900x over previous
import functools

import jax
import jax.numpy as jnp
from jax import lax
from jax.experimental import pallas as pl
from jax.experimental.pallas import tpu as pltpu
from jax.experimental.pallas import tpu_sc as plsc

EMB = 32
BATCH = 16384
CH = 128
NBUF = 4


def _make_kernel(ngroups, batch):
    info = plsc.get_sparse_core_info()
    nw = info.num_cores * info.num_subcores
    b_per_w = batch // nw
    nch = (2 * b_per_w) // CH
    mesh = plsc.VectorSubcoreMesh(core_axis_name="c", subcore_axis_name="s")

    @functools.partial(
        pl.kernel,
        mesh=mesh,
        out_type=[
            jax.ShapeDtypeStruct((batch, EMB), jnp.float32),
            jax.ShapeDtypeStruct((batch, EMB), jnp.float32),
        ],
        scratch_types=[
            pltpu.VMEM((2 * b_per_w,), jnp.int32),
        ]
        + [pltpu.VMEM((CH // 8, 8, EMB), jnp.float32) for _ in range(NBUF)]
        + [pltpu.SemaphoreType.DMA for _ in range(NBUF)]
        + [pltpu.SemaphoreType.DMA for _ in range(NBUF)],
        compiler_params=pltpu.CompilerParams(needs_layout_passes=False),
    )
    def gather_kernel(table_hbm, uidx_hbm, iidx_hbm, out_u, out_i,
                      idx_v, *bufs_sems):
        bufs = bufs_sems[:NBUF]
        gsems = bufs_sems[NBUF:2 * NBUF]
        wsems = bufs_sems[2 * NBUF:]
        wid = lax.axis_index("s") * info.num_cores + lax.axis_index("c")
        base = wid * b_per_w
        pltpu.sync_copy(uidx_hbm.at[pl.ds(base, b_per_w)],
                        idx_v.at[pl.ds(0, b_per_w)])
        pltpu.sync_copy(iidx_hbm.at[pl.ds(base, b_per_w)],
                        idx_v.at[pl.ds(b_per_w, b_per_w)])

        def fire_gather(c, buf, gsem):
            def blk(kb, _):
                v = idx_v[pl.ds(c * CH + kb * 16, 16)]
                for j in range(16):
                    pltpu.async_copy(
                        table_hbm.at[pl.ds(v[j] >> 3, 1), pl.ds(v[j] & 7, 1)],
                        buf.at[pl.ds(kb * 2 + j // 8, 1), pl.ds(j % 8, 1)],
                        gsem)
                return 0

            lax.fori_loop(0, CH // 16, blk, 0)

        def wait_gather(buf, gsem):
            pltpu.make_async_copy(
                table_hbm.at[pl.ds(0, CH // 8)], buf, gsem).wait()

        def out_row0(c):
            return base + (c % (nch // 2)) * CH

        def fire_writeout(c, buf, wsem):
            out = out_u if c < nch // 2 else out_i
            row0 = out_row0(c)

            def row(k, _):
                pltpu.async_copy(buf.at[k // 8, k % 8],
                                 out.at[row0 + k], wsem)
                return 0

            lax.fori_loop(0, CH, row, 0)

        def wait_writeout(c, buf, wsem):
            out = out_u if c < nch // 2 else out_i
            pltpu.make_async_copy(buf, out.at[pl.ds(out_row0(c), CH)],
                                  wsem).wait()

        for c in range(NBUF):
            fire_gather(c, bufs[c], gsems[c])
        for c in range(nch):
            p = c % NBUF
            wait_gather(bufs[p], gsems[p])
            fire_writeout(c, bufs[p], wsems[p])
            if c + NBUF < nch:
                wait_writeout(c, bufs[p], wsems[p])
                fire_gather(c + NBUF, bufs[p], gsems[p])
        for c in range(max(nch - NBUF, 0), nch):
            p = c % NBUF
            wait_writeout(c, bufs[p], wsems[p])

    return gather_kernel


def kernel(embeddings, user_ids, item_ids):
    vocab, emb = embeddings.shape
    table3 = embeddings.reshape(vocab // 8, 8, emb)
    batch = user_ids.shape[0]
    fn = _make_kernel(vocab // 8, batch)
    users_emb, items_emb = fn(table3, user_ids, item_ids)
    return (users_emb, items_emb)
